# Initial kernel scaffold; baseline (speedup 1.0000x reference)
import jax
import jax.numpy as jnp
from jax import lax
from jax.experimental import pallas as pl
from jax.experimental.pallas import tpu as pltpu
from jax.experimental.pallas import tpu_sc as plsc

N = 10000
E = 320000
D = 128

NC = 2
NS = 16
NW = NC * NS
EDGES_PER_W = E // NW
CHUNK = 80
CHUNKS_PER_W = EDGES_PER_W // CHUNK
ROWS_PER_TILE = N // NS
ZBLK = 125
DEG_ZBLK = 1000


def _make_sc_kernel():
    mesh = plsc.VectorSubcoreMesh(core_axis_name="c", subcore_axis_name="s",
                                  num_cores=NC, num_subcores=NS)

    def body(x_hbm, src_hbm, dst_hbm, z2_hbm, z1_hbm, ones_hbm,
             agg_hbm, deg_hbm,
             src_v, dst_v, sbuf, dbuf, rows_v, ones_v, agg_sh, deg_sh, sem):
        c = lax.axis_index("c")
        s = lax.axis_index("s")
        wid = s * NC + c

        pltpu.sync_copy(src_hbm.at[wid], src_v)
        pltpu.sync_copy(dst_hbm.at[wid], dst_v)
        pltpu.sync_copy(ones_hbm, ones_v)

        row0 = s * ROWS_PER_TILE
        for k in range(ROWS_PER_TILE // ZBLK):
            pltpu.sync_copy(z2_hbm, agg_sh.at[pl.ds(row0 + k * ZBLK, ZBLK)])

        @pl.when(s == 0)
        def _zero_deg():
            for k in range(N // DEG_ZBLK):
                pltpu.sync_copy(z1_hbm, deg_sh.at[pl.ds(k * DEG_ZBLK,
                                                        DEG_ZBLK)])

        plsc.subcore_barrier()

        @pl.loop(0, CHUNKS_PER_W)
        def _edge_chunk(j):
            pltpu.sync_copy(src_v.at[j], sbuf)
            pltpu.sync_copy(dst_v.at[j], dbuf)
            pltpu.async_copy(x_hbm.at[sbuf], rows_v, sem).wait()
            pltpu.sync_copy(rows_v, agg_sh.at[dbuf], add=True)
            pltpu.sync_copy(ones_v, deg_sh.at[dbuf], add=True)

        plsc.subcore_barrier()

        pltpu.sync_copy(agg_sh.at[pl.ds(row0, ROWS_PER_TILE)],
                        agg_hbm.at[c, pl.ds(row0, ROWS_PER_TILE)])

        @pl.when(s == 0)
        def _deg_out():
            pltpu.sync_copy(deg_sh, deg_hbm.at[c])

    return pl.kernel(
        body,
        out_type=(
            jax.ShapeDtypeStruct((NC, N, D), jnp.float32),
            jax.ShapeDtypeStruct((NC, N), jnp.float32),
        ),
        mesh=mesh,
        scratch_types=[
            pltpu.VMEM((CHUNKS_PER_W, CHUNK), jnp.int32),
            pltpu.VMEM((CHUNKS_PER_W, CHUNK), jnp.int32),
            pltpu.VMEM((CHUNK,), jnp.int32),
            pltpu.VMEM((CHUNK,), jnp.int32),
            pltpu.VMEM((CHUNK, D), jnp.float32),
            pltpu.VMEM((CHUNK,), jnp.float32),
            pltpu.VMEM_SHARED((N, D), jnp.float32),
            pltpu.VMEM_SHARED((N,), jnp.float32),
            pltpu.SemaphoreType.DMA,
        ],
        name="sage_scatter_sc",
    )


_sc_kernel = _make_sc_kernel()

BLK = 2000


def _tc_body(x_ref, a0_ref, a1_ref, d0_ref, d1_ref, wl_ref, wr_ref, b_ref,
             o_ref):
    deg = d0_ref[...] + d1_ref[...]
    inv = 1.0 / jnp.maximum(deg, 1.0)
    mean = (a0_ref[...] + a1_ref[...]) * inv
    x = x_ref[...]
    node_emb = (jnp.dot(mean, wl_ref[...], preferred_element_type=jnp.float32)
                + b_ref[...]
                + jnp.dot(x, wr_ref[...], preferred_element_type=jnp.float32))
    o_ref[...] = 0.5 * (x + node_emb)


@jax.jit
def kernel(x, edge_index, W_l, b_l, W_r):
    src = edge_index[0].reshape(NW, CHUNKS_PER_W, CHUNK)
    dst = edge_index[1].reshape(NW, CHUNKS_PER_W, CHUNK)
    zeros2d = jnp.zeros((ZBLK, D), jnp.float32)
    zeros1d = jnp.zeros((DEG_ZBLK,), jnp.float32)
    ones = jnp.ones((CHUNK,), jnp.float32)

    agg, deg = _sc_kernel(x, src, dst, zeros2d, zeros1d, ones)

    row_spec = pl.BlockSpec((BLK, D), lambda i: (i, 0))
    deg_spec = pl.BlockSpec((BLK, 1), lambda i: (i, 0))
    mat_spec = pl.BlockSpec((D, D), lambda i: (0, 0))
    bias_spec = pl.BlockSpec((1, D), lambda i: (0, 0))

    out = pl.pallas_call(
        _tc_body,
        grid=(N // BLK,),
        in_specs=[row_spec, row_spec, row_spec, deg_spec, deg_spec,
                  mat_spec, mat_spec, bias_spec],
        out_specs=row_spec,
        out_shape=jax.ShapeDtypeStruct((N, D), jnp.float32),
    )(x, agg[0], agg[1], deg[0, :, None], deg[1, :, None],
      W_l.T, W_r.T, b_l[None, :])
    return out


# speedup vs baseline: 7.8279x; 7.8279x over previous
import jax
import jax.numpy as jnp
from jax import lax
from jax.experimental import pallas as pl
from jax.experimental.pallas import tpu as pltpu
from jax.experimental.pallas import tpu_sc as plsc

N = 10000
E = 320000
D = 128

NC = 2
NS = 16
NW = NC * NS
EDGES_PER_W = E // NW
CHUNK = 80
CHUNKS_PER_W = EDGES_PER_W // CHUNK
N_PAD = 10240
ROWS_PER_TILE = N_PAD // NS
ZBLK = 128
DEG_ZBLK = 1024


def _make_sc_kernel():
    mesh = plsc.VectorSubcoreMesh(core_axis_name="c", subcore_axis_name="s",
                                  num_cores=NC, num_subcores=NS)

    def body(x_hbm, src_hbm, dst_hbm, z2_hbm, z1_hbm, ones_hbm,
             agg_hbm, deg_hbm,
             src_v, dst_v, rows_v, ones_v, agg_sh, deg_sh, sem):
        c = lax.axis_index("c")
        s = lax.axis_index("s")
        wid = s * NC + c

        pltpu.sync_copy(src_hbm.at[wid], src_v)
        pltpu.sync_copy(dst_hbm.at[wid], dst_v)
        pltpu.sync_copy(ones_hbm, ones_v)

        row0 = s * ROWS_PER_TILE
        for k in range(ROWS_PER_TILE // ZBLK):
            pltpu.sync_copy(z2_hbm, agg_sh.at[pl.ds(row0 + k * ZBLK, ZBLK)])

        @pl.when(s == 0)
        def _zero_deg():
            for k in range(N_PAD // DEG_ZBLK):
                pltpu.sync_copy(z1_hbm, deg_sh.at[pl.ds(k * DEG_ZBLK,
                                                        DEG_ZBLK)])

        plsc.subcore_barrier()

        @pl.loop(0, CHUNKS_PER_W)
        def _edge_chunk(j):
            pltpu.async_copy(x_hbm.at[src_v.at[j]], rows_v, sem).wait()
            pltpu.sync_copy(rows_v, agg_sh.at[dst_v.at[j]], add=True)
            pltpu.sync_copy(ones_v, deg_sh.at[dst_v.at[j]], add=True)

        plsc.subcore_barrier()

        pltpu.sync_copy(agg_sh.at[pl.ds(row0, ROWS_PER_TILE)],
                        agg_hbm.at[c, pl.ds(row0, ROWS_PER_TILE)])

        @pl.when(s == 0)
        def _deg_out():
            pltpu.sync_copy(deg_sh, deg_hbm.at[c])

    return pl.kernel(
        body,
        out_type=(
            jax.ShapeDtypeStruct((NC, N_PAD, D), jnp.float32),
            jax.ShapeDtypeStruct((NC, N_PAD), jnp.float32),
        ),
        mesh=mesh,
        scratch_types=[
            pltpu.VMEM((CHUNKS_PER_W, CHUNK), jnp.int32),
            pltpu.VMEM((CHUNKS_PER_W, CHUNK), jnp.int32),
            pltpu.VMEM((CHUNK, D), jnp.float32),
            pltpu.VMEM((CHUNK,), jnp.float32),
            pltpu.VMEM_SHARED((N_PAD, D), jnp.float32),
            pltpu.VMEM_SHARED((N_PAD,), jnp.float32),
            pltpu.SemaphoreType.DMA,
        ],
        name="sage_scatter_sc",
    )


_sc_kernel = _make_sc_kernel()

BLK = 2000


def _tc_body(x_ref, a0_ref, a1_ref, d0_ref, d1_ref, wl_ref, wr_ref, b_ref,
             o_ref):
    deg = d0_ref[...] + d1_ref[...]
    inv = 1.0 / jnp.maximum(deg, 1.0)
    mean = (a0_ref[...] + a1_ref[...]) * inv
    x = x_ref[...]
    node_emb = (jnp.dot(mean, wl_ref[...], preferred_element_type=jnp.float32)
                + b_ref[...]
                + jnp.dot(x, wr_ref[...], preferred_element_type=jnp.float32))
    o_ref[...] = 0.5 * (x + node_emb)


@jax.jit
def kernel(x, edge_index, W_l, b_l, W_r):
    src = edge_index[0].reshape(NW, CHUNKS_PER_W, CHUNK)
    dst = edge_index[1].reshape(NW, CHUNKS_PER_W, CHUNK)
    zeros2d = jnp.zeros((ZBLK, D), jnp.float32)
    zeros1d = jnp.zeros((DEG_ZBLK,), jnp.float32)
    ones = jnp.ones((CHUNK,), jnp.float32)

    agg, deg = _sc_kernel(x, src, dst, zeros2d, zeros1d, ones)

    row_spec = pl.BlockSpec((BLK, D), lambda i: (i, 0))
    deg_spec = pl.BlockSpec((BLK, 1), lambda i: (i, 0))
    mat_spec = pl.BlockSpec((D, D), lambda i: (0, 0))
    bias_spec = pl.BlockSpec((1, D), lambda i: (0, 0))

    out = pl.pallas_call(
        _tc_body,
        grid=(N // BLK,),
        in_specs=[row_spec, row_spec, row_spec, deg_spec, deg_spec,
                  mat_spec, mat_spec, bias_spec],
        out_specs=row_spec,
        out_shape=jax.ShapeDtypeStruct((N, D), jnp.float32),
    )(x, agg[0], agg[1], deg[0, :, None], deg[1, :, None],
      W_l.T, W_r.T, b_l[None, :])
    return out
